# Initial kernel scaffold; baseline (speedup 1.0000x reference)
import jax
import jax.numpy as jnp
from jax.experimental import pallas as pl


def kernel(mana_token, type_token, mana_table, type_table, W, b):
    raise NotImplementedError("write your pallas kernel here")


# speedup vs baseline: 2.8008x; 2.8008x over previous
import functools
import jax
import jax.numpy as jnp
from jax import lax
from jax.experimental import pallas as pl
from jax.experimental.pallas import tpu as pltpu
from jax.experimental.pallas import tpu_sc as plsc

EMBED = 128
BATCH = 16384

_info = plsc.get_sparse_core_info()
_NC, _NS = _info.num_cores, _info.num_subcores
_NW = _NC * _NS
_B_PER_W = BATCH // _NW
_CHUNK = 128
_NCHUNK = _B_PER_W // _CHUNK


@functools.partial(
    pl.kernel,
    out_type=[
        jax.ShapeDtypeStruct((BATCH, EMBED), jnp.float32),
        jax.ShapeDtypeStruct((BATCH, EMBED), jnp.float32),
    ],
    scratch_types=[
        pltpu.VMEM((_CHUNK,), jnp.int32),
        pltpu.VMEM((_CHUNK, EMBED), jnp.float32),
        pltpu.SemaphoreType.DMA,
    ],
    mesh=plsc.VectorSubcoreMesh(core_axis_name="c", subcore_axis_name="s"),
)
def _sc_gather(mana_idx, type_idx, mana_tab, type_tab, out1, out2,
               idx_v, rows_v, sem):
    wid = lax.axis_index("s") * _NC + lax.axis_index("c")
    base = wid * _B_PER_W
    for idx_hbm, tab, out in ((mana_idx, mana_tab, out1),
                              (type_idx, type_tab, out2)):
        for c in range(_NCHUNK):
            off = base + c * _CHUNK
            pltpu.sync_copy(idx_hbm.at[pl.ds(off, _CHUNK)], idx_v)
            pltpu.async_copy(tab.at[idx_v], rows_v, sem).wait()
            pltpu.sync_copy(rows_v, out.at[pl.ds(off, _CHUNK)])


def _mm_body(g1_ref, g2_ref, w_ref, b_ref, o_ref):
    w1 = w_ref[:EMBED, :]
    w2 = w_ref[EMBED:, :]
    acc = jnp.dot(g1_ref[...], w1, preferred_element_type=jnp.float32)
    acc += jnp.dot(g2_ref[...], w2, preferred_element_type=jnp.float32)
    o_ref[...] = acc + b_ref[...]


_TM = 2048


@jax.jit
def _tc_matmul(g1, g2, W, b2d):
    return pl.pallas_call(
        _mm_body,
        grid=(BATCH // _TM,),
        in_specs=[
            pl.BlockSpec((_TM, EMBED), lambda i: (i, 0)),
            pl.BlockSpec((_TM, EMBED), lambda i: (i, 0)),
            pl.BlockSpec((2 * EMBED, EMBED), lambda i: (0, 0)),
            pl.BlockSpec((1, EMBED), lambda i: (0, 0)),
        ],
        out_specs=pl.BlockSpec((_TM, EMBED), lambda i: (i, 0)),
        out_shape=jax.ShapeDtypeStruct((BATCH, EMBED), jnp.float32),
    )(g1, g2, W, b2d)


@jax.jit
def kernel(mana_token, type_token, mana_table, type_table, W, b):
    g1, g2 = _sc_gather(mana_token.astype(jnp.int32),
                        type_token.astype(jnp.int32),
                        mana_table, type_table)
    return _tc_matmul(g1, g2, W, b.reshape(1, EMBED))


# speedup vs baseline: 3.3429x; 1.1935x over previous
import functools
import jax
import jax.numpy as jnp
from jax import lax
from jax.experimental import pallas as pl
from jax.experimental.pallas import tpu as pltpu
from jax.experimental.pallas import tpu_sc as plsc

EMBED = 128
BATCH = 16384

_info = plsc.get_sparse_core_info()
_NC, _NS = _info.num_cores, _info.num_subcores
_NW = _NC * _NS
_B_PER_W = BATCH // _NW
_CHUNK = 128
_NCHUNK = _B_PER_W // _CHUNK


_NBUF = 4
_NCHUNK_TOT = 2 * _NCHUNK


@functools.partial(
    pl.kernel,
    out_type=[
        jax.ShapeDtypeStruct((BATCH, EMBED), jnp.float32),
        jax.ShapeDtypeStruct((BATCH, EMBED), jnp.float32),
    ],
    scratch_types=(
        [pltpu.VMEM((_NCHUNK_TOT, _CHUNK), jnp.int32)]
        + [pltpu.VMEM((_CHUNK, EMBED), jnp.float32) for _ in range(_NBUF)]
        + [pltpu.SemaphoreType.DMA for _ in range(2 * _NBUF)]
    ),
    mesh=plsc.VectorSubcoreMesh(core_axis_name="c", subcore_axis_name="s"),
)
def _sc_gather(mana_idx, type_idx, mana_tab, type_tab, out1, out2,
               idx_v, *bufs_and_sems):
    bufs = bufs_and_sems[:_NBUF]
    gsem = bufs_and_sems[_NBUF:2 * _NBUF]
    ssem = bufs_and_sems[2 * _NBUF:]
    wid = lax.axis_index("s") * _NC + lax.axis_index("c")
    base = wid * _B_PER_W

    pltpu.sync_copy(mana_idx.at[pl.ds(wid * _NCHUNK, _NCHUNK)],
                    idx_v.at[pl.ds(0, _NCHUNK)])
    pltpu.sync_copy(type_idx.at[pl.ds(wid * _NCHUNK, _NCHUNK)],
                    idx_v.at[pl.ds(_NCHUNK, _NCHUNK)])

    def tab_of(c):
        return mana_tab if c < _NCHUNK else type_tab

    def out_of(c):
        return out1 if c < _NCHUNK else out2

    def off_of(c):
        return base + (c % _NCHUNK) * _CHUNK

    gd = [None] * _NCHUNK_TOT
    sd = [None] * _NCHUNK_TOT
    for c in range(_NBUF):
        gd[c] = pltpu.async_copy(tab_of(c).at[idx_v.at[c]], bufs[c], gsem[c])
    for c in range(_NCHUNK_TOT):
        b = c % _NBUF
        gd[c].wait()
        sd[c] = pltpu.async_copy(bufs[b], out_of(c).at[pl.ds(off_of(c), _CHUNK)],
                                 ssem[b])
        if c + _NBUF < _NCHUNK_TOT:
            sd[c].wait()
            gd[c + _NBUF] = pltpu.async_copy(
                tab_of(c + _NBUF).at[idx_v.at[c + _NBUF]], bufs[b], gsem[b])
    for c in range(_NCHUNK_TOT - _NBUF, _NCHUNK_TOT):
        sd[c].wait()


def _mm_body(g1_ref, g2_ref, w_ref, b_ref, o_ref):
    w1 = w_ref[:EMBED, :]
    w2 = w_ref[EMBED:, :]
    acc = jnp.dot(g1_ref[...], w1, preferred_element_type=jnp.float32)
    acc += jnp.dot(g2_ref[...], w2, preferred_element_type=jnp.float32)
    o_ref[...] = acc + b_ref[...]


_TM = 2048


@jax.jit
def _tc_matmul(g1, g2, W, b2d):
    return pl.pallas_call(
        _mm_body,
        grid=(BATCH // _TM,),
        in_specs=[
            pl.BlockSpec((_TM, EMBED), lambda i: (i, 0)),
            pl.BlockSpec((_TM, EMBED), lambda i: (i, 0)),
            pl.BlockSpec((2 * EMBED, EMBED), lambda i: (0, 0)),
            pl.BlockSpec((1, EMBED), lambda i: (0, 0)),
        ],
        out_specs=pl.BlockSpec((_TM, EMBED), lambda i: (i, 0)),
        out_shape=jax.ShapeDtypeStruct((BATCH, EMBED), jnp.float32),
    )(g1, g2, W, b2d)


@jax.jit
def kernel(mana_token, type_token, mana_table, type_table, W, b):
    g1, g2 = _sc_gather(mana_token.astype(jnp.int32).reshape(BATCH // _CHUNK, _CHUNK),
                        type_token.astype(jnp.int32).reshape(BATCH // _CHUNK, _CHUNK),
                        mana_table, type_table)
    return _tc_matmul(g1, g2, W, b.reshape(1, EMBED))


# speedup vs baseline: 3.3756x; 1.0098x over previous
import functools
import jax
import jax.numpy as jnp
from jax import lax
from jax.experimental import pallas as pl
from jax.experimental.pallas import tpu as pltpu
from jax.experimental.pallas import tpu_sc as plsc

EMBED = 128
BATCH = 16384

_info = plsc.get_sparse_core_info()
_NC, _NS = _info.num_cores, _info.num_subcores
_NW = _NC * _NS
_B_PER_W = BATCH // _NW
_CHUNK = 128
_NCHUNK = _B_PER_W // _CHUNK


_NBUF = 7
_NCHUNK_TOT = 2 * _NCHUNK


@functools.partial(
    pl.kernel,
    out_type=[
        jax.ShapeDtypeStruct((BATCH, EMBED), jnp.float32),
        jax.ShapeDtypeStruct((BATCH, EMBED), jnp.float32),
    ],
    scratch_types=(
        [pltpu.VMEM((_NCHUNK_TOT, _CHUNK), jnp.int32)]
        + [pltpu.VMEM((_CHUNK, EMBED), jnp.float32) for _ in range(_NBUF)]
        + [pltpu.SemaphoreType.DMA for _ in range(2 * _NBUF)]
    ),
    mesh=plsc.VectorSubcoreMesh(core_axis_name="c", subcore_axis_name="s"),
)
def _sc_gather(mana_idx, type_idx, mana_tab, type_tab, out1, out2,
               idx_v, *bufs_and_sems):
    bufs = bufs_and_sems[:_NBUF]
    gsem = bufs_and_sems[_NBUF:2 * _NBUF]
    ssem = bufs_and_sems[2 * _NBUF:]
    wid = lax.axis_index("s") * _NC + lax.axis_index("c")
    base = wid * _B_PER_W

    pltpu.sync_copy(mana_idx.at[pl.ds(wid * _NCHUNK, _NCHUNK)],
                    idx_v.at[pl.ds(0, _NCHUNK)])
    pltpu.sync_copy(type_idx.at[pl.ds(wid * _NCHUNK, _NCHUNK)],
                    idx_v.at[pl.ds(_NCHUNK, _NCHUNK)])

    def tab_of(c):
        return mana_tab if c < _NCHUNK else type_tab

    def out_of(c):
        return out1 if c < _NCHUNK else out2

    def off_of(c):
        return base + (c % _NCHUNK) * _CHUNK

    gd = [None] * _NCHUNK_TOT
    sd = [None] * _NCHUNK_TOT
    for c in range(_NBUF):
        gd[c] = pltpu.async_copy(tab_of(c).at[idx_v.at[c]], bufs[c], gsem[c])
    for c in range(_NCHUNK_TOT):
        b = c % _NBUF
        gd[c].wait()
        sd[c] = pltpu.async_copy(bufs[b], out_of(c).at[pl.ds(off_of(c), _CHUNK)],
                                 ssem[b])
        if c + _NBUF < _NCHUNK_TOT:
            sd[c].wait()
            gd[c + _NBUF] = pltpu.async_copy(
                tab_of(c + _NBUF).at[idx_v.at[c + _NBUF]], bufs[b], gsem[b])
    for c in range(_NCHUNK_TOT - _NBUF, _NCHUNK_TOT):
        sd[c].wait()


def _mm_body(g1_ref, g2_ref, w_ref, b_ref, o_ref):
    w1 = w_ref[:EMBED, :]
    w2 = w_ref[EMBED:, :]
    acc = jnp.dot(g1_ref[...], w1, preferred_element_type=jnp.float32)
    acc += jnp.dot(g2_ref[...], w2, preferred_element_type=jnp.float32)
    o_ref[...] = acc + b_ref[...]


_TM = 2048


@jax.jit
def _tc_matmul(g1, g2, W, b2d):
    return pl.pallas_call(
        _mm_body,
        grid=(BATCH // _TM,),
        in_specs=[
            pl.BlockSpec((_TM, EMBED), lambda i: (i, 0)),
            pl.BlockSpec((_TM, EMBED), lambda i: (i, 0)),
            pl.BlockSpec((2 * EMBED, EMBED), lambda i: (0, 0)),
            pl.BlockSpec((1, EMBED), lambda i: (0, 0)),
        ],
        out_specs=pl.BlockSpec((_TM, EMBED), lambda i: (i, 0)),
        out_shape=jax.ShapeDtypeStruct((BATCH, EMBED), jnp.float32),
    )(g1, g2, W, b2d)


@jax.jit
def kernel(mana_token, type_token, mana_table, type_table, W, b):
    g1, g2 = _sc_gather(mana_token.astype(jnp.int32).reshape(BATCH // _CHUNK, _CHUNK),
                        type_token.astype(jnp.int32).reshape(BATCH // _CHUNK, _CHUNK),
                        mana_table, type_table)
    return _tc_matmul(g1, g2, W, b.reshape(1, EMBED))
